# Initial kernel scaffold; baseline (speedup 1.0000x reference)
import jax
import jax.numpy as jnp
from jax.experimental import pallas as pl


def kernel(dense, sparse, W, b):
    raise NotImplementedError("write your pallas kernel here")


# speedup vs baseline: 19.8372x; 19.8372x over previous
import functools
import itertools

import jax
import jax.numpy as jnp
from jax import lax
from jax.experimental import pallas as pl
from jax.experimental.pallas import tpu as pltpu
from jax.experimental.pallas import tpu_sc as plsc

DENSE_SIZE = 13
OUT_DIM = 16
BATCH = 1024
N_SPARSE = 8
CARD = 50
PAIRS = list(itertools.combinations(range(N_SPARSE), 2))
N_FEAT = N_SPARSE + len(PAIRS)
OFF_SINGLE = [DENSE_SIZE + CARD * i for i in range(N_SPARSE)]
OFF_PAIR = [DENSE_SIZE + CARD * N_SPARSE + CARD * CARD * p
            for p in range(len(PAIRS))]
FEATURE_DIM = DENSE_SIZE + N_SPARSE * CARD + len(PAIRS) * CARD * CARD

NC, NS, L = 2, 16, 16
NW = NC * NS
ROWS_PER_W = BATCH // NW
IDX_PER_W = ROWS_PER_W * N_FEAT
GCHUNK = 128
NG = IDX_PER_W // GCHUNK


def _sc_body(wt_hbm, sparse_hbm, dense_hbm, b_hbm, out_hbm,
             sparse_v, dense_v, b_v, wd_v, idx_v, gat_v, out_v, sem):
    wid = lax.axis_index("s") * NC + lax.axis_index("c")
    row0 = wid * ROWS_PER_W

    pltpu.sync_copy(
        sparse_hbm.at[pl.ds(row0 * N_SPARSE, ROWS_PER_W * N_SPARSE)],
        sparse_v)
    pltpu.sync_copy(
        dense_hbm.at[pl.ds(row0 * DENSE_SIZE, ROWS_PER_W * DENSE_SIZE)],
        dense_v.at[pl.ds(8, ROWS_PER_W * DENSE_SIZE)])
    pltpu.sync_copy(b_hbm, b_v)
    pltpu.sync_copy(wt_hbm.at[pl.ds(0, L)], wd_v)

    lanes = lax.iota(jnp.int32, L)
    for h in range(ROWS_PER_W // L):
        cols = [
            plsc.load_gather(
                sparse_v, [lanes * N_SPARSE + (h * L * N_SPARSE + i)])
            for i in range(N_SPARSE)
        ]
        for k in range(N_SPARSE):
            idx_v[pl.ds(k * ROWS_PER_W + h * L, L)] = cols[k] + OFF_SINGLE[k]
        for p, (i, j) in enumerate(PAIRS):
            idx_v[pl.ds((N_SPARSE + p) * ROWS_PER_W + h * L, L)] = (
                cols[i] * CARD + cols[j] + OFF_PAIR[p])

    copies = [
        pltpu.async_copy(
            wt_hbm.at[idx_v.at[pl.ds(g * GCHUNK, GCHUNK)]],
            gat_v.at[pl.ds(g * GCHUNK, GCHUNK)], sem)
        for g in range(NG)
    ]
    for c in copies:
        c.wait()

    for r in range(ROWS_PER_W):
        acc = b_v[...]
        for k in range(N_FEAT):
            acc = acc + gat_v[k * ROWS_PER_W + r]
        for d in range(DENSE_SIZE):
            s = plsc.load_gather(
                dense_v, [jnp.full((L,), 8 + r * DENSE_SIZE + d, jnp.int32)])
            acc = acc + s * wd_v[d]
        out_v[r] = jnp.maximum(acc, 0.0)

    pltpu.sync_copy(out_v, out_hbm.at[pl.ds(row0, ROWS_PER_W)])


def kernel(dense, sparse, W, b):
    wt = W.T
    run = pl.kernel(
        _sc_body,
        out_type=jax.ShapeDtypeStruct((BATCH, OUT_DIM), jnp.float32),
        mesh=plsc.VectorSubcoreMesh(
            core_axis_name="c", subcore_axis_name="s"),
        compiler_params=pltpu.CompilerParams(
            needs_layout_passes=False, use_tc_tiling_on_sc=False),
        scratch_types=[
            pltpu.VMEM((ROWS_PER_W * N_SPARSE,), jnp.int32),
            pltpu.VMEM((8 + ROWS_PER_W * DENSE_SIZE,), jnp.float32),
            pltpu.VMEM((OUT_DIM,), jnp.float32),
            pltpu.VMEM((L, OUT_DIM), jnp.float32),
            pltpu.VMEM((IDX_PER_W,), jnp.int32),
            pltpu.VMEM((IDX_PER_W, OUT_DIM), jnp.float32),
            pltpu.VMEM((ROWS_PER_W, OUT_DIM), jnp.float32),
            pltpu.SemaphoreType.DMA,
        ],
    )
    return run(wt, sparse.reshape(-1), dense.reshape(-1), b)
